# baseline (speedup 1.0000x reference)
import dataclasses
import functools

import jax
import jax.numpy as jnp
from jax import lax
from jax.experimental import pallas as pl
from jax.experimental.pallas import tpu as pltpu
from jax.experimental.pallas import tpu_sc as plsc

GAMMA_ = 12.0
N_ = 10000
E_ = 320000
D_ = 128
NC_ = 2
NS_ = 16
L_ = 16
NW_ = NC_ * NS_
EPT_ = E_ // NW_
B_ = 40
NBLK_ = EPT_ // B_
NPAIR_ = NBLK_ // 2
RPT_ = 624
REM_ = N_ - NS_ * RPT_
BP_ = 48


def _edge_block_compute(gv, relv, msgv, d2v, scv):
    @pl.loop(0, B_, step=2)
    def _(e0):
        for e in (e0, e0 + 1):
            accs = [jnp.zeros((L_,), jnp.float32) for _ in range(4)]
            for j in range(D_ // L_):
                h = gv[e, pl.ds(L_ * j, L_)]
                r = relv[e, pl.ds(L_ * j, L_)]
                t = gv[B_ + e, pl.ds(L_ * j, L_)]
                tr = h + r
                d = tr - t
                accs[j % 4] = accs[j % 4] + d * d
                msgv[e, pl.ds(L_ * j, L_)] = tr
            acc = (accs[0] + accs[1]) + (accs[2] + accs[3])
            d2v[pl.ds(e * L_, L_)] = jnp.broadcast_to(jnp.sum(acc), (L_,))

    for j in range(BP_ // L_):
        offs = jnp.arange(16, dtype=jnp.int32) * L_ + (L_ * L_) * j
        d2 = plsc.load_gather(d2v, [offs])
        bits = lax.bitcast_convert_type(d2, jnp.int32)
        seed = jnp.full((L_,), 0x5F3759DF, jnp.int32) - (bits >> 1)
        y = lax.bitcast_convert_type(seed, jnp.float32)
        half = d2 * 0.5
        y = y * (1.5 - half * y * y)
        y = y * (1.5 - half * y * y)
        dist = d2 * y
        scv[pl.ds(L_ * j, L_)] = 1.0 / (1.0 + jnp.exp(dist - GAMMA_))

    @pl.loop(0, B_, step=2)
    def _(e0):
        for e in (e0, e0 + 1):
            s = plsc.load_gather(scv, [jnp.broadcast_to(e, (L_,))])
            for j in range(D_ // L_):
                msgv[e, pl.ds(L_ * j, L_)] = msgv[e, pl.ds(L_ * j, L_)] * s


def _sc_partials(x, idx5, rel, zrows):
    mesh = plsc.VectorSubcoreMesh(core_axis_name="c", subcore_axis_name="s")
    cp = pltpu.CompilerParams()
    if "needs_layout_passes" in pltpu.CompilerParams.__dataclass_fields__:
        cp = dataclasses.replace(cp, needs_layout_passes=False)

    @functools.partial(
        pl.kernel,
        compiler_params=cp,
        out_type=jax.ShapeDtypeStruct((NC_ * N_, D_), jnp.float32),
        mesh=mesh,
        scratch_types=[
            pltpu.VMEM((2, 2 * B_), jnp.int32),
            pltpu.VMEM((2, 2 * B_), jnp.int32),
            pltpu.VMEM((2, B_), jnp.int32),
            pltpu.VMEM((2, B_), jnp.int32),
            pltpu.VMEM((2 * B_, D_), jnp.float32),
            pltpu.VMEM((2 * B_, D_), jnp.float32),
            pltpu.VMEM((B_, D_), jnp.float32),
            pltpu.VMEM((B_, D_), jnp.float32),
            pltpu.VMEM((B_, D_), jnp.float32),
            pltpu.VMEM((BP_ * L_,), jnp.float32),
            pltpu.VMEM((BP_,), jnp.float32),
            pltpu.VMEM_SHARED((N_, D_), jnp.float32),
            pltpu.SemaphoreType.DMA,
            pltpu.SemaphoreType.DMA,
            pltpu.SemaphoreType.DMA,
            pltpu.SemaphoreType.DMA,
        ],
    )
    def k(x_hbm, idxg_hbm, idxs_hbm, rel_hbm, z_hbm, out_hbm,
          idxgA, idxgB, idxsA, idxsB, gv0, gv1, relv0, relv1,
          msgv, d2v, scv, hsh, semd0, semd1, semiA, semiB):
        cid = lax.axis_index("c")
        sid = lax.axis_index("s")
        wid = sid * NC_ + cid
        gv = (gv0, gv1)
        relv = (relv0, relv1)
        semd = (semd0, semd1)
        idxG = (idxgA, idxgB)
        idxS = (idxsA, idxsB)
        semi = (semiA, semiB)

        pltpu.sync_copy(z_hbm, hsh.at[pl.ds(sid * RPT_, RPT_)])

        @pl.when(sid == 0)
        def _():
            pltpu.sync_copy(z_hbm.at[pl.ds(0, REM_)],
                            hsh.at[pl.ds(NS_ * RPT_, REM_)])

        plsc.subcore_barrier()

        def issue_idx(p, ab):
            pltpu.async_copy(idxg_hbm.at[wid, p], idxG[ab], semi[ab])
            pltpu.async_copy(idxs_hbm.at[wid, p], idxS[ab], semi[ab])

        def wait_idx(ab):
            pltpu.make_async_copy(idxg_hbm.at[wid, 0], idxG[ab],
                                  semi[ab]).wait()
            pltpu.make_async_copy(idxs_hbm.at[wid, 0], idxS[ab],
                                  semi[ab]).wait()

        def issue2(b, buf, ab, row):
            pltpu.async_copy(x_hbm.at[idxG[ab].at[row]], gv[buf], semd[buf])
            pltpu.async_copy(rel_hbm.at[pl.ds(wid * EPT_ + b * B_, B_)],
                             relv[buf], semd[buf])

        def wait2(buf):
            pltpu.make_async_copy(rel_hbm.at[pl.ds(0, 2 * B_)], gv[buf],
                                  semd[buf]).wait()
            pltpu.make_async_copy(rel_hbm.at[pl.ds(0, B_)], relv[buf],
                                  semd[buf]).wait()

        def step(b, buf, ab, row):
            wait2(buf)
            _edge_block_compute(gv[buf], relv[buf], msgv, d2v, scv)
            pltpu.sync_copy(msgv, hsh.at[idxS[ab].at[row]], add=True)

        pltpu.sync_copy(idxg_hbm.at[wid, 0], idxgA)
        pltpu.sync_copy(idxs_hbm.at[wid, 0], idxsA)
        issue2(0, 0, 0, 0)
        issue_idx(1, 1)

        @pl.loop(0, (NPAIR_ - 1) // 2)
        def _(kk):
            b0 = 4 * kk
            issue2(b0 + 1, 1, 0, 1)
            step(b0, 0, 0, 0)
            wait_idx(1)
            issue2(b0 + 2, 0, 1, 0)
            step(b0 + 1, 1, 0, 1)
            issue_idx(2 * kk + 2, 0)
            issue2(b0 + 3, 1, 1, 1)
            step(b0 + 2, 0, 1, 0)
            wait_idx(0)
            issue2(b0 + 4, 0, 0, 0)
            step(b0 + 3, 1, 1, 1)

            @pl.when(kk < (NPAIR_ - 1) // 2 - 1)
            def _():
                issue_idx(2 * kk + 3, 1)

        issue2(NBLK_ - 1, 1, 0, 1)
        step(NBLK_ - 2, 0, 0, 0)
        step(NBLK_ - 1, 1, 0, 1)

        plsc.subcore_barrier()
        pltpu.sync_copy(
            hsh.at[pl.ds(sid * RPT_, RPT_)],
            out_hbm.at[pl.ds(cid * N_ + sid * RPT_, RPT_)],
        )

        @pl.when(sid == 0)
        def _():
            pltpu.sync_copy(
                hsh.at[pl.ds(NS_ * RPT_, REM_)],
                out_hbm.at[pl.ds(cid * N_ + NS_ * RPT_, REM_)],
            )

    return k(x, idx5[0], idx5[1], rel, zrows)


def _combine(partials):
    bn = 2000

    def add_k(p_ref, o_ref):
        o_ref[...] = p_ref[0] + p_ref[1]

    return pl.pallas_call(
        add_k,
        out_shape=jax.ShapeDtypeStruct((N_, D_), jnp.float32),
        grid=(N_ // bn,),
        in_specs=[pl.BlockSpec((2, bn, D_), lambda i: (0, i, 0))],
        out_specs=pl.BlockSpec((bn, D_), lambda i: (i, 0)),
    )(partials)


@jax.jit
def kernel(x, edge_index, edge_attr):
    ei = edge_index.astype(jnp.int32).reshape(2, NW_, NPAIR_, 2, B_)
    idxg = ei.transpose(1, 2, 3, 0, 4).reshape(NW_, NPAIR_, 2, 2 * B_)
    idxs = ei[1]
    zrows = jnp.zeros((RPT_, D_), jnp.float32)
    partials = _sc_partials(x, (idxg, idxs), edge_attr, zrows)
    return _combine(partials.reshape(NC_, N_, D_))


# speedup vs baseline: 1.5768x; 1.5768x over previous
import dataclasses
import functools

import jax
import jax.numpy as jnp
from jax import lax
from jax.experimental import pallas as pl
from jax.experimental.pallas import tpu as pltpu
from jax.experimental.pallas import tpu_sc as plsc

GAMMA_ = 12.0
N_ = 10000
E_ = 320000
D_ = 128
NC_ = 2
NS_ = 16
L_ = 16
NW_ = NC_ * NS_
EPT_ = E_ // NW_
B_ = 40
NBLK_ = EPT_ // B_
NPAIR_ = NBLK_ // 2
RPT_ = 624
REM_ = N_ - NS_ * RPT_
BP_ = 48


def _edge_block_compute(gv, relv, msgv):
    @pl.loop(0, B_, step=2)
    def _(e0):
        for e in (e0, e0 + 1):
            accs = [jnp.zeros((L_,), jnp.float32) for _ in range(4)]
            trs = []
            for j in range(D_ // L_):
                h = gv[e, pl.ds(L_ * j, L_)]
                r = relv[e, pl.ds(L_ * j, L_)]
                t = gv[B_ + e, pl.ds(L_ * j, L_)]
                tr = h + r
                d = tr - t
                accs[j % 4] = accs[j % 4] + d * d
                trs.append(tr)
            acc = (accs[0] + accs[1]) + (accs[2] + accs[3])
            d2 = jnp.broadcast_to(jnp.sum(acc), (L_,))
            bits = lax.bitcast_convert_type(d2, jnp.int32)
            seed = jnp.full((L_,), 0x5F3759DF, jnp.int32) - (bits >> 1)
            y = lax.bitcast_convert_type(seed, jnp.float32)
            half = d2 * 0.5
            y = y * (1.5 - half * y * y)
            y = y * (1.5 - half * y * y)
            dist = d2 * y
            score = 1.0 / (1.0 + jnp.exp(dist - GAMMA_))
            for j in range(D_ // L_):
                msgv[e, pl.ds(L_ * j, L_)] = trs[j] * score


def _sc_partials(x, idx5, rel, zrows):
    mesh = plsc.VectorSubcoreMesh(core_axis_name="c", subcore_axis_name="s")
    cp = pltpu.CompilerParams()
    if "needs_layout_passes" in pltpu.CompilerParams.__dataclass_fields__:
        cp = dataclasses.replace(cp, needs_layout_passes=False)

    @functools.partial(
        pl.kernel,
        compiler_params=cp,
        out_type=jax.ShapeDtypeStruct((NC_ * N_, D_), jnp.float32),
        mesh=mesh,
        scratch_types=[
            pltpu.VMEM((2, 2 * B_), jnp.int32),
            pltpu.VMEM((2, 2 * B_), jnp.int32),
            pltpu.VMEM((2, B_), jnp.int32),
            pltpu.VMEM((2, B_), jnp.int32),
            pltpu.VMEM((2 * B_, D_), jnp.float32),
            pltpu.VMEM((2 * B_, D_), jnp.float32),
            pltpu.VMEM((B_, D_), jnp.float32),
            pltpu.VMEM((B_, D_), jnp.float32),
            pltpu.VMEM((B_, D_), jnp.float32),
            pltpu.VMEM_SHARED((N_, D_), jnp.float32),
            pltpu.SemaphoreType.DMA,
            pltpu.SemaphoreType.DMA,
            pltpu.SemaphoreType.DMA,
            pltpu.SemaphoreType.DMA,
        ],
    )
    def k(x_hbm, idxg_hbm, idxs_hbm, rel_hbm, z_hbm, out_hbm,
          idxgA, idxgB, idxsA, idxsB, gv0, gv1, relv0, relv1,
          msgv, hsh, semd0, semd1, semiA, semiB):
        cid = lax.axis_index("c")
        sid = lax.axis_index("s")
        wid = sid * NC_ + cid
        gv = (gv0, gv1)
        relv = (relv0, relv1)
        semd = (semd0, semd1)
        idxG = (idxgA, idxgB)
        idxS = (idxsA, idxsB)
        semi = (semiA, semiB)

        pltpu.sync_copy(z_hbm, hsh.at[pl.ds(sid * RPT_, RPT_)])

        @pl.when(sid == 0)
        def _():
            pltpu.sync_copy(z_hbm.at[pl.ds(0, REM_)],
                            hsh.at[pl.ds(NS_ * RPT_, REM_)])

        plsc.subcore_barrier()

        def issue_idx(p, ab):
            pltpu.async_copy(idxg_hbm.at[wid, p], idxG[ab], semi[ab])
            pltpu.async_copy(idxs_hbm.at[wid, p], idxS[ab], semi[ab])

        def wait_idx(ab):
            pltpu.make_async_copy(idxg_hbm.at[wid, 0], idxG[ab],
                                  semi[ab]).wait()
            pltpu.make_async_copy(idxs_hbm.at[wid, 0], idxS[ab],
                                  semi[ab]).wait()

        def issue2(b, buf, ab, row):
            pltpu.async_copy(x_hbm.at[idxG[ab].at[row]], gv[buf], semd[buf])
            pltpu.async_copy(rel_hbm.at[pl.ds(wid * EPT_ + b * B_, B_)],
                             relv[buf], semd[buf])

        def wait2(buf):
            pltpu.make_async_copy(rel_hbm.at[pl.ds(0, 2 * B_)], gv[buf],
                                  semd[buf]).wait()
            pltpu.make_async_copy(rel_hbm.at[pl.ds(0, B_)], relv[buf],
                                  semd[buf]).wait()

        def step(b, buf, ab, row):
            wait2(buf)
            _edge_block_compute(gv[buf], relv[buf], msgv)
            pltpu.sync_copy(msgv, hsh.at[idxS[ab].at[row]], add=True)

        pltpu.sync_copy(idxg_hbm.at[wid, 0], idxgA)
        pltpu.sync_copy(idxs_hbm.at[wid, 0], idxsA)
        issue2(0, 0, 0, 0)
        issue_idx(1, 1)

        @pl.loop(0, (NPAIR_ - 1) // 2)
        def _(kk):
            b0 = 4 * kk
            issue2(b0 + 1, 1, 0, 1)
            step(b0, 0, 0, 0)
            wait_idx(1)
            issue2(b0 + 2, 0, 1, 0)
            step(b0 + 1, 1, 0, 1)
            issue_idx(2 * kk + 2, 0)
            issue2(b0 + 3, 1, 1, 1)
            step(b0 + 2, 0, 1, 0)
            wait_idx(0)
            issue2(b0 + 4, 0, 0, 0)
            step(b0 + 3, 1, 1, 1)

            @pl.when(kk < (NPAIR_ - 1) // 2 - 1)
            def _():
                issue_idx(2 * kk + 3, 1)

        issue2(NBLK_ - 1, 1, 0, 1)
        step(NBLK_ - 2, 0, 0, 0)
        step(NBLK_ - 1, 1, 0, 1)

        plsc.subcore_barrier()
        pltpu.sync_copy(
            hsh.at[pl.ds(sid * RPT_, RPT_)],
            out_hbm.at[pl.ds(cid * N_ + sid * RPT_, RPT_)],
        )

        @pl.when(sid == 0)
        def _():
            pltpu.sync_copy(
                hsh.at[pl.ds(NS_ * RPT_, REM_)],
                out_hbm.at[pl.ds(cid * N_ + NS_ * RPT_, REM_)],
            )

    return k(x, idx5[0], idx5[1], rel, zrows)


def _combine(partials):
    bn = 2000

    def add_k(p_ref, o_ref):
        o_ref[...] = p_ref[0] + p_ref[1]

    return pl.pallas_call(
        add_k,
        out_shape=jax.ShapeDtypeStruct((N_, D_), jnp.float32),
        grid=(N_ // bn,),
        in_specs=[pl.BlockSpec((2, bn, D_), lambda i: (0, i, 0))],
        out_specs=pl.BlockSpec((bn, D_), lambda i: (i, 0)),
    )(partials)


@jax.jit
def kernel(x, edge_index, edge_attr):
    ei = edge_index.astype(jnp.int32).reshape(2, NW_, NPAIR_, 2, B_)
    idxg = ei.transpose(1, 2, 3, 0, 4).reshape(NW_, NPAIR_, 2, 2 * B_)
    idxs = ei[1]
    zrows = jnp.zeros((RPT_, D_), jnp.float32)
    partials = _sc_partials(x, (idxg, idxs), edge_attr, zrows)
    return _combine(partials.reshape(NC_, N_, D_))


# speedup vs baseline: 1.7135x; 1.0866x over previous
import dataclasses
import functools

import jax
import jax.numpy as jnp
from jax import lax
from jax.experimental import pallas as pl
from jax.experimental.pallas import tpu as pltpu
from jax.experimental.pallas import tpu_sc as plsc

GAMMA_ = 12.0
N_ = 10000
E_ = 320000
D_ = 128
NC_ = 2
NS_ = 16
L_ = 16
NW_ = NC_ * NS_
EPT_ = E_ // NW_
B_ = 40
NBLK_ = EPT_ // B_
RPT_ = 624
REM_ = N_ - NS_ * RPT_
NRING_ = 4


def _edge_block_compute(gv, relv, msgv):
    @pl.loop(0, B_, step=2)
    def _(e0):
        for e in (e0, e0 + 1):
            accs = [jnp.zeros((L_,), jnp.float32) for _ in range(4)]
            trs = []
            for j in range(D_ // L_):
                h = gv[e, pl.ds(L_ * j, L_)]
                r = relv[e, pl.ds(L_ * j, L_)]
                t = gv[B_ + e, pl.ds(L_ * j, L_)]
                tr = h + r
                d = tr - t
                accs[j % 4] = accs[j % 4] + d * d
                trs.append(tr)
            acc = (accs[0] + accs[1]) + (accs[2] + accs[3])
            d2 = jnp.broadcast_to(jnp.sum(acc), (L_,))
            bits = lax.bitcast_convert_type(d2, jnp.int32)
            seed = jnp.full((L_,), 0x5F3759DF, jnp.int32) - (bits >> 1)
            y = lax.bitcast_convert_type(seed, jnp.float32)
            half = d2 * 0.5
            y = y * (1.5 - half * y * y)
            y = y * (1.5 - half * y * y)
            dist = d2 * y
            score = 1.0 / (1.0 + jnp.exp(dist - GAMMA_))
            for j in range(D_ // L_):
                msgv[e, pl.ds(L_ * j, L_)] = trs[j] * score


def _sc_partials(x, idxg, idxs, rel, zrows):
    mesh = plsc.VectorSubcoreMesh(core_axis_name="c", subcore_axis_name="s")
    cp = pltpu.CompilerParams()
    if "needs_layout_passes" in pltpu.CompilerParams.__dataclass_fields__:
        cp = dataclasses.replace(cp, needs_layout_passes=False)

    @functools.partial(
        pl.kernel,
        compiler_params=cp,
        out_type=jax.ShapeDtypeStruct((NC_ * N_, D_), jnp.float32),
        mesh=mesh,
        scratch_types=[
            pltpu.VMEM((NRING_, 1, 2 * B_), jnp.int32),
            pltpu.VMEM((NRING_, 1, B_), jnp.int32),
            pltpu.VMEM((2 * B_, D_), jnp.float32),
            pltpu.VMEM((2 * B_, D_), jnp.float32),
            pltpu.VMEM((B_, D_), jnp.float32),
            pltpu.VMEM((B_, D_), jnp.float32),
            pltpu.VMEM((B_, D_), jnp.float32),
            pltpu.VMEM_SHARED((N_, D_), jnp.float32),
            pltpu.SemaphoreType.DMA,
            pltpu.SemaphoreType.DMA,
            pltpu.SemaphoreType.DMA,
            pltpu.SemaphoreType.DMA,
        ],
    )
    def k(x_hbm, idxg_hbm, idxs_hbm, rel_hbm, z_hbm, out_hbm,
          ringG, ringS, gv0, gv1, relv0, relv1, msgv, hsh,
          semd0, semd1, semi, sems):
        cid = lax.axis_index("c")
        sid = lax.axis_index("s")
        wid = sid * NC_ + cid
        gv = (gv0, gv1)
        relv = (relv0, relv1)
        semd = (semd0, semd1)

        pltpu.sync_copy(z_hbm, hsh.at[pl.ds(sid * RPT_, RPT_)])

        @pl.when(sid == 0)
        def _():
            pltpu.sync_copy(z_hbm.at[pl.ds(0, REM_)],
                            hsh.at[pl.ds(NS_ * RPT_, REM_)])

        plsc.subcore_barrier()

        def issue_idx(b):
            slot = lax.rem(b, NRING_)
            pltpu.async_copy(idxg_hbm.at[wid, b], ringG.at[slot], semi)
            pltpu.async_copy(idxs_hbm.at[wid, b], ringS.at[slot], semi)

        def wait_idx():
            pltpu.make_async_copy(idxg_hbm.at[wid, 0], ringG.at[0],
                                  semi).wait()
            pltpu.make_async_copy(idxs_hbm.at[wid, 0], ringS.at[0],
                                  semi).wait()

        def issue2(b, buf):
            slot = lax.rem(b, NRING_)
            pltpu.async_copy(x_hbm.at[ringG.at[slot, 0]], gv[buf], semd[buf])
            pltpu.async_copy(rel_hbm.at[pl.ds(wid * EPT_ + b * B_, B_)],
                             relv[buf], semd[buf])

        def wait2(buf):
            pltpu.make_async_copy(rel_hbm.at[pl.ds(0, 2 * B_)], gv[buf],
                                  semd[buf]).wait()
            pltpu.make_async_copy(rel_hbm.at[pl.ds(0, B_)], relv[buf],
                                  semd[buf]).wait()

        def wait_scatter():
            pltpu.make_async_copy(msgv, hsh.at[pl.ds(0, B_)], sems).wait()

        def step(b, buf):
            wait2(buf)

            @pl.when(b > 0)
            def _():
                wait_scatter()

            _edge_block_compute(gv[buf], relv[buf], msgv)
            slot = lax.rem(b, NRING_)
            pltpu.async_copy(msgv, hsh.at[ringS.at[slot, 0]], sems, add=True)

            @pl.when(b + 2 < NBLK_)
            def _():
                wait_idx()
                issue2(b + 2, buf)

            @pl.when(b + 3 < NBLK_)
            def _():
                issue_idx(b + 3)

        pltpu.sync_copy(idxg_hbm.at[wid, 0], ringG.at[0])
        pltpu.sync_copy(idxs_hbm.at[wid, 0], ringS.at[0])
        pltpu.sync_copy(idxg_hbm.at[wid, 1], ringG.at[1])
        pltpu.sync_copy(idxs_hbm.at[wid, 1], ringS.at[1])
        issue2(0, 0)
        issue2(1, 1)
        issue_idx(2)

        @pl.loop(0, NBLK_ // 2)
        def _(i):
            step(2 * i, 0)
            step(2 * i + 1, 1)

        wait_scatter()
        plsc.subcore_barrier()
        pltpu.sync_copy(
            hsh.at[pl.ds(sid * RPT_, RPT_)],
            out_hbm.at[pl.ds(cid * N_ + sid * RPT_, RPT_)],
        )

        @pl.when(sid == 0)
        def _():
            pltpu.sync_copy(
                hsh.at[pl.ds(NS_ * RPT_, REM_)],
                out_hbm.at[pl.ds(cid * N_ + NS_ * RPT_, REM_)],
            )

    return k(x, idxg, idxs, rel, zrows)


def _combine(partials):
    bn = 2000

    def add_k(p_ref, o_ref):
        o_ref[...] = p_ref[0] + p_ref[1]

    return pl.pallas_call(
        add_k,
        out_shape=jax.ShapeDtypeStruct((N_, D_), jnp.float32),
        grid=(N_ // bn,),
        in_specs=[pl.BlockSpec((2, bn, D_), lambda i: (0, i, 0))],
        out_specs=pl.BlockSpec((bn, D_), lambda i: (i, 0)),
    )(partials)


@jax.jit
def kernel(x, edge_index, edge_attr):
    ei = edge_index.astype(jnp.int32).reshape(2, NW_, NBLK_, 1, B_)
    idxg = ei.transpose(1, 2, 3, 0, 4).reshape(NW_, NBLK_, 1, 2 * B_)
    idxs = ei[1]
    zrows = jnp.zeros((RPT_, D_), jnp.float32)
    partials = _sc_partials(x, idxg, idxs, edge_attr, zrows)
    return _combine(partials.reshape(NC_, N_, D_))
